# baseline (speedup 1.0000x reference)
import functools

import jax
import jax.numpy as jnp
from jax import lax
from jax.experimental import pallas as pl
from jax.experimental.pallas import tpu as pltpu
from jax.experimental.pallas import tpu_sc as plsc

BATCH = 4096
HIST_LEN = 200
EMBED_DIM = 64
VOCAB = 1000000

NUM_CORES = 2
NUM_SUBCORES = 16
NUM_WORKERS = NUM_CORES * NUM_SUBCORES

TOTAL = BATCH * HIST_LEN
B_PER_W = TOTAL // NUM_WORKERS
CHUNK = 640
NCHUNK = B_PER_W // CHUNK
SUB = 2
G = CHUNK // SUB


def _embed_kernel(table_hbm, idx_hbm, out_hbm, idx_v, rows_v,
                  gsem0, gsem1, osem0, osem1):
  wid = lax.axis_index("s") * NUM_CORES + lax.axis_index("c")
  base = wid * B_PER_W
  gsems = (gsem0, gsem1)
  osems = (osem0, osem1)

  def fire_gather(g, b):
    off = base + g * CHUNK
    pltpu.sync_copy(idx_hbm.at[pl.ds(off, CHUNK)], idx_v.at[b])
    for j in range(SUB):
      pltpu.async_copy(
          table_hbm.at[idx_v.at[b, pl.ds(j * G, G)]],
          rows_v.at[b, pl.ds(j * G, G)],
          gsems[b],
      )

  def wait_gather(b):
    pltpu.make_async_copy(table_hbm.at[idx_v.at[b]], rows_v.at[b],
                          gsems[b]).wait()

  def fire_out(g, b):
    off = base + g * CHUNK
    pltpu.async_copy(rows_v.at[b],
                     out_hbm.at[pl.ds(off, CHUNK), pl.ds(0, EMBED_DIM)],
                     osems[b])

  def wait_out(b):
    pltpu.make_async_copy(rows_v.at[b],
                          out_hbm.at[pl.ds(base, CHUNK), pl.ds(0, EMBED_DIM)],
                          osems[b]).wait()

  fire_gather(0, 0)
  fire_gather(1, 1)

  @pl.loop(0, NCHUNK, step=2)
  def _step(g):
    for b in range(2):
      gg = g + b
      wait_gather(b)
      fire_out(gg, b)

      @pl.when(gg + 2 < NCHUNK)
      def _refill():
        wait_out(b)
        fire_gather(gg + 2, b)

  wait_out(0)
  wait_out(1)


def kernel(input_ids, table):
  idx2 = input_ids.reshape(TOTAL).astype(jnp.int32) * 2
  table_p = jnp.pad(table, ((0, 0), (0, 64))).reshape(2 * VOCAB, EMBED_DIM)
  mesh = plsc.VectorSubcoreMesh(core_axis_name="c", subcore_axis_name="s")
  run = functools.partial(
      pl.kernel,
      mesh=mesh,
      out_type=jax.ShapeDtypeStruct((TOTAL, 128), jnp.float32),
      scratch_types=[
          pltpu.VMEM((2, CHUNK), jnp.int32),
          pltpu.VMEM((2, CHUNK, EMBED_DIM), jnp.float32),
          pltpu.SemaphoreType.DMA,
          pltpu.SemaphoreType.DMA,
          pltpu.SemaphoreType.DMA,
          pltpu.SemaphoreType.DMA,
      ],
      compiler_params=pltpu.CompilerParams(use_tc_tiling_on_sc=False),
  )(_embed_kernel)
  out = run(table_p, idx2)
  return out[:, :EMBED_DIM].reshape(BATCH, HIST_LEN, EMBED_DIM)


# speedup vs baseline: 1.0100x; 1.0100x over previous
import functools

import jax
import jax.numpy as jnp
from jax import lax
from jax.experimental import pallas as pl
from jax.experimental.pallas import tpu as pltpu
from jax.experimental.pallas import tpu_sc as plsc

BATCH = 4096
HIST_LEN = 200
EMBED_DIM = 64
VOCAB = 1000000

NUM_CORES = 2
NUM_SUBCORES = 16
NUM_WORKERS = NUM_CORES * NUM_SUBCORES

TOTAL = BATCH * HIST_LEN
B_PER_W = TOTAL // NUM_WORKERS
CHUNK = 640
NCHUNK = B_PER_W // CHUNK
SUB = 2
G = CHUNK // SUB


def _embed_kernel(table_hbm, idx_hbm, out_hbm, idx_v, rows_v,
                  gsem0, gsem1, osem0, osem1):
  wid = lax.axis_index("s") * NUM_CORES + lax.axis_index("c")
  base = wid * B_PER_W
  gsems = (gsem0, gsem1)
  osems = (osem0, osem1)

  pltpu.sync_copy(idx_hbm.at[pl.ds(base, B_PER_W)], idx_v)

  def fire_gather(g, b):
    for j in range(SUB):
      pltpu.async_copy(
          table_hbm.at[idx_v.at[pl.ds(g * CHUNK + j * G, G)]],
          rows_v.at[b, pl.ds(j * G, G)],
          gsems[b],
      )

  def wait_gather(b):
    pltpu.make_async_copy(table_hbm.at[idx_v.at[pl.ds(0, CHUNK)]],
                          rows_v.at[b], gsems[b]).wait()

  def fire_out(g, b):
    off = base + g * CHUNK
    pltpu.async_copy(rows_v.at[b],
                     out_hbm.at[pl.ds(off, CHUNK), pl.ds(0, EMBED_DIM)],
                     osems[b])

  def wait_out(b):
    pltpu.make_async_copy(rows_v.at[b],
                          out_hbm.at[pl.ds(base, CHUNK), pl.ds(0, EMBED_DIM)],
                          osems[b]).wait()

  fire_gather(0, 0)
  fire_gather(1, 1)

  @pl.loop(0, NCHUNK, step=2)
  def _step(g):
    for b in range(2):
      gg = g + b
      wait_gather(b)
      fire_out(gg, b)

      @pl.when(gg + 2 < NCHUNK)
      def _refill():
        wait_out(b)
        fire_gather(gg + 2, b)

  wait_out(0)
  wait_out(1)


def kernel(input_ids, table):
  idx2 = input_ids.reshape(TOTAL).astype(jnp.int32) * 2
  table_p = jnp.pad(table, ((0, 0), (0, 64))).reshape(2 * VOCAB, EMBED_DIM)
  mesh = plsc.VectorSubcoreMesh(core_axis_name="c", subcore_axis_name="s")
  run = functools.partial(
      pl.kernel,
      mesh=mesh,
      out_type=jax.ShapeDtypeStruct((TOTAL, 128), jnp.float32),
      scratch_types=[
          pltpu.VMEM((B_PER_W,), jnp.int32),
          pltpu.VMEM((2, CHUNK, EMBED_DIM), jnp.float32),
          pltpu.SemaphoreType.DMA,
          pltpu.SemaphoreType.DMA,
          pltpu.SemaphoreType.DMA,
          pltpu.SemaphoreType.DMA,
      ],
      compiler_params=pltpu.CompilerParams(use_tc_tiling_on_sc=False),
  )(_embed_kernel)
  out = run(table_p, idx2)
  return out[:, :EMBED_DIM].reshape(BATCH, HIST_LEN, EMBED_DIM)


# speedup vs baseline: 1.0102x; 1.0002x over previous
import functools

import jax
import jax.numpy as jnp
from jax import lax
from jax.experimental import pallas as pl
from jax.experimental.pallas import tpu as pltpu
from jax.experimental.pallas import tpu_sc as plsc

BATCH = 4096
HIST_LEN = 200
EMBED_DIM = 64
VOCAB = 1000000

NUM_CORES = 2
NUM_SUBCORES = 16
NUM_WORKERS = NUM_CORES * NUM_SUBCORES

TOTAL = BATCH * HIST_LEN
B_PER_W = TOTAL // NUM_WORKERS
CHUNK = 320
NCHUNK = B_PER_W // CHUNK
SUB = 1
G = CHUNK // SUB
NBUF = 4


def _embed_kernel(table_hbm, idx_hbm, out_hbm, idx_v, rows_v,
                  gsem0, gsem1, gsem2, gsem3, osem0, osem1, osem2, osem3):
  wid = lax.axis_index("s") * NUM_CORES + lax.axis_index("c")
  base = wid * B_PER_W
  gsems = (gsem0, gsem1, gsem2, gsem3)
  osems = (osem0, osem1, osem2, osem3)

  pltpu.sync_copy(idx_hbm.at[pl.ds(base, B_PER_W)], idx_v)

  def fire_gather(g, b):
    for j in range(SUB):
      pltpu.async_copy(
          table_hbm.at[idx_v.at[pl.ds(g * CHUNK + j * G, G)]],
          rows_v.at[b, pl.ds(j * G, G)],
          gsems[b],
      )

  def wait_gather(b):
    pltpu.make_async_copy(table_hbm.at[idx_v.at[pl.ds(0, CHUNK)]],
                          rows_v.at[b], gsems[b]).wait()

  def fire_out(g, b):
    off = base + g * CHUNK
    pltpu.async_copy(rows_v.at[b],
                     out_hbm.at[pl.ds(off, CHUNK), pl.ds(0, EMBED_DIM)],
                     osems[b])

  def wait_out(b):
    pltpu.make_async_copy(rows_v.at[b],
                          out_hbm.at[pl.ds(base, CHUNK), pl.ds(0, EMBED_DIM)],
                          osems[b]).wait()

  for b in range(NBUF):
    fire_gather(b, b)

  @pl.loop(0, NCHUNK, step=NBUF)
  def _step(g):
    for b in range(NBUF):
      gg = g + b
      wait_gather(b)
      fire_out(gg, b)

      @pl.when(gg + NBUF < NCHUNK)
      def _refill():
        wait_out(b)
        fire_gather(gg + NBUF, b)

  for b in range(NBUF):
    wait_out(b)


def kernel(input_ids, table):
  idx2 = input_ids.reshape(TOTAL).astype(jnp.int32) * 2
  table_p = jnp.pad(table, ((0, 0), (0, 64))).reshape(2 * VOCAB, EMBED_DIM)
  mesh = plsc.VectorSubcoreMesh(core_axis_name="c", subcore_axis_name="s")
  run = functools.partial(
      pl.kernel,
      mesh=mesh,
      out_type=jax.ShapeDtypeStruct((TOTAL, 128), jnp.float32),
      scratch_types=[
          pltpu.VMEM((B_PER_W,), jnp.int32),
          pltpu.VMEM((NBUF, CHUNK, EMBED_DIM), jnp.float32),
          pltpu.SemaphoreType.DMA,
          pltpu.SemaphoreType.DMA,
          pltpu.SemaphoreType.DMA,
          pltpu.SemaphoreType.DMA,
          pltpu.SemaphoreType.DMA,
          pltpu.SemaphoreType.DMA,
          pltpu.SemaphoreType.DMA,
          pltpu.SemaphoreType.DMA,
      ],
      compiler_params=pltpu.CompilerParams(use_tc_tiling_on_sc=False),
  )(_embed_kernel)
  out = run(table_p, idx2)
  return out[:, :EMBED_DIM].reshape(BATCH, HIST_LEN, EMBED_DIM)
